# baseline (speedup 1.0000x reference)
import functools

import jax
import jax.numpy as jnp
from jax import lax
from jax.experimental import pallas as pl
from jax.experimental.pallas import tpu as pltpu
from jax.experimental.pallas import tpu_sc as plsc

N_SPARSE = 26
N_DENSE = 13
VOCAB = 100000
DIM = 32
B = 4096
OUT_W = N_SPARSE * DIM + N_DENSE

NC = 2
NS = 16
NW = NC * NS
BPW = B // NW
IPW = BPW * N_SPARSE


def _make_sc_embed():
    mesh = plsc.VectorSubcoreMesh(core_axis_name="c", subcore_axis_name="s")

    @functools.partial(
        pl.kernel,
        mesh=mesh,
        out_type=jax.ShapeDtypeStruct((B, OUT_W), jnp.float32),
        scratch_types=[
            pltpu.VMEM((IPW,), jnp.int32),
            pltpu.VMEM((IPW, DIM), jnp.float32),
            pltpu.VMEM((BPW, N_DENSE), jnp.float32),
            pltpu.SemaphoreType.DMA,
        ],
        compiler_params=pltpu.CompilerParams(use_tc_tiling_on_sc=False),
    )
    def sc_embed(dense_hbm, idx_hbm, tables_hbm, out_hbm,
                 idx_v, emb_v, dense_v, sem):
        wid = lax.axis_index("s") * NC + lax.axis_index("c")
        base = wid * BPW
        pltpu.sync_copy(idx_hbm.at[pl.ds(wid * IPW, IPW)], idx_v)
        pltpu.sync_copy(dense_hbm.at[pl.ds(base, BPW)], dense_v)
        copies = []
        for f in range(N_SPARSE):
            copies.append(pltpu.async_copy(
                tables_hbm.at[f].at[idx_v.at[pl.ds(f * BPW, BPW)]],
                emb_v.at[pl.ds(f * BPW, BPW)],
                sem))
        for cp in copies:
            cp.wait()
        for f in range(N_SPARSE):
            pltpu.sync_copy(
                emb_v.at[pl.ds(f * BPW, BPW)],
                out_hbm.at[pl.ds(base, BPW), pl.ds(f * DIM, DIM)])
        pltpu.sync_copy(
            dense_v, out_hbm.at[pl.ds(base, BPW), pl.ds(N_SPARSE * DIM, N_DENSE)])

    return sc_embed


def kernel(x_dense, x_sparse, tables):
    idx_fm = x_sparse.reshape(NW, BPW, N_SPARSE).transpose(0, 2, 1).reshape(-1)
    return _make_sc_embed()(x_dense, idx_fm, tables)


# speedup vs baseline: 1.6904x; 1.6904x over previous
import functools

import jax
import jax.numpy as jnp
from jax import lax
from jax.experimental import pallas as pl
from jax.experimental.pallas import tpu as pltpu
from jax.experimental.pallas import tpu_sc as plsc

N_SPARSE = 26
N_DENSE = 13
VOCAB = 100000
DIM = 32
B = 4096
EMB_W = N_SPARSE * DIM
OUT_W = EMB_W + N_DENSE

NC = 2
NS = 16
NW = NC * NS
BPW = B // NW
RPB = 16
NBLK = BPW // RPB


def _make_sc_embed():
    mesh = plsc.VectorSubcoreMesh(core_axis_name="c", subcore_axis_name="s")

    @functools.partial(
        pl.kernel,
        mesh=mesh,
        out_type=jax.ShapeDtypeStruct((B, OUT_W), jnp.float32),
        scratch_types=[
            pltpu.VMEM((RPB * EMB_W,), jnp.int32),
            pltpu.VMEM((BPW, EMB_W), jnp.float32),
            pltpu.VMEM((BPW, N_DENSE), jnp.float32),
            pltpu.SemaphoreType.DMA,
        ],
        compiler_params=pltpu.CompilerParams(use_tc_tiling_on_sc=False),
    )
    def sc_embed(dense_hbm, eidx_hbm, tables_hbm, out_hbm,
                 idx_v, asm_v, dense_v, sem):
        wid = lax.axis_index("s") * NC + lax.axis_index("c")
        base = wid * BPW
        pltpu.sync_copy(dense_hbm.at[pl.ds(base, BPW)], dense_v)

        def blk_body(blk, _):
            row0 = base + blk * RPB
            pltpu.sync_copy(eidx_hbm.at[pl.ds(row0 * EMB_W, RPB * EMB_W)],
                            idx_v)
            copies = []
            for rr in range(RPB):
                copies.append(pltpu.async_copy(
                    tables_hbm.at[idx_v.at[pl.ds(rr * EMB_W, EMB_W)]],
                    asm_v.at[blk * RPB + rr],
                    sem))
            for cp in copies:
                cp.wait()
            return 0

        lax.fori_loop(0, NBLK, blk_body, 0)
        pltpu.sync_copy(asm_v, out_hbm.at[pl.ds(base, BPW), pl.ds(0, EMB_W)])
        pltpu.sync_copy(dense_v,
                        out_hbm.at[pl.ds(base, BPW), pl.ds(EMB_W, N_DENSE)])

    return sc_embed


def kernel(x_dense, x_sparse, tables):
    offs = (jnp.arange(N_SPARSE, dtype=jnp.int32) * (DIM * VOCAB))[:, None] \
        + (jnp.arange(DIM, dtype=jnp.int32) * VOCAB)[None, :]
    eidx = (x_sparse[:, :, None] + offs[None, :, :]).reshape(-1)
    tables_e = jnp.transpose(tables, (0, 2, 1)).reshape(-1)
    return _make_sc_embed()(x_dense, eidx, tables_e)
